# Initial kernel scaffold; baseline (speedup 1.0000x reference)
import jax
import jax.numpy as jnp
from jax.experimental import pallas as pl


def kernel(x, edge_index, W, b):
    raise NotImplementedError("write your pallas kernel here")


# speedup vs baseline: 42.9483x; 42.9483x over previous
import functools

import jax
import jax.numpy as jnp
from jax import lax
from jax.experimental import pallas as pl
from jax.experimental.pallas import tpu as pltpu
from jax.experimental.pallas import tpu_sc as plsc

N_NODES = 10000
N_EDGES = 320000
IN_C = 128
D = 16
NPAD = 10240
SLAB = NPAD // 16
CHUNK = 128
CPT = 80
NTILES = 32
CT = CPT * NTILES
EPAD = CT * CHUNK

_mesh = plsc.VectorSubcoreMesh(core_axis_name="c", subcore_axis_name="s")
_sc_params = pltpu.CompilerParams(use_tc_tiling_on_sc=False)


@functools.partial(
    pl.kernel,
    mesh=_mesh,
    out_type=jax.ShapeDtypeStruct((2 * NPAD,), jnp.float32),
    compiler_params=_sc_params,
    scratch_types=[
        pltpu.VMEM((CPT, CHUNK), jnp.int32),
        pltpu.VMEM((CHUNK,), jnp.float32),
        pltpu.VMEM((SLAB,), jnp.float32),
        pltpu.VMEM_SHARED((NPAD,), jnp.float32),
    ],
)
def _sc_degree(dst_hbm, out_hbm, idx_v, ones_v, stage_v, hist_s):
    c = lax.axis_index("c")
    s = lax.axis_index("s")
    w = c * 16 + s

    for i in range(CHUNK // 16):
        ones_v[pl.ds(i * 16, 16)] = jnp.ones((16,), jnp.float32)
    for i in range(SLAB // 16):
        stage_v[pl.ds(i * 16, 16)] = jnp.zeros((16,), jnp.float32)

    pltpu.sync_copy(stage_v, hist_s.at[pl.ds(s * SLAB, SLAB)])
    plsc.subcore_barrier()

    pltpu.sync_copy(dst_hbm.at[pl.ds(w * CPT, CPT)], idx_v)

    def body(j, carry):
        pltpu.sync_copy(ones_v, hist_s.at[idx_v.at[j]], add=True)
        return carry

    lax.fori_loop(0, CPT, body, 0)
    plsc.subcore_barrier()

    pltpu.sync_copy(hist_s.at[pl.ds(s * SLAB, SLAB)], stage_v)
    pltpu.sync_copy(stage_v, out_hbm.at[pl.ds(c * NPAD + s * SLAB, SLAB)])


@functools.partial(
    pl.kernel,
    mesh=_mesh,
    out_type=jax.ShapeDtypeStruct((2 * NPAD, D), jnp.float32),
    compiler_params=_sc_params,
    scratch_types=[
        pltpu.VMEM((CPT, CHUNK), jnp.int32),
        pltpu.VMEM((CPT, CHUNK), jnp.int32),
        pltpu.VMEM((CHUNK, D), jnp.float32),
        pltpu.VMEM((CHUNK, D), jnp.float32),
        pltpu.VMEM_SHARED((NPAD, D), jnp.float32),
        pltpu.SemaphoreType.DMA,
    ],
)
def _sc_aggregate(y_hbm, src_hbm, dst_hbm, out_hbm,
                  isrc, idst, rows, stage, agg_s, sem):
    c = lax.axis_index("c")
    s = lax.axis_index("s")
    w = c * 16 + s

    for i in range(CHUNK):
        stage[i] = jnp.zeros((16,), jnp.float32)
    for k in range(SLAB // CHUNK):
        pltpu.sync_copy(stage, agg_s.at[pl.ds(s * SLAB + k * CHUNK, CHUNK)])
    plsc.subcore_barrier()

    pltpu.sync_copy(src_hbm.at[pl.ds(w * CPT, CPT)], isrc)
    pltpu.sync_copy(dst_hbm.at[pl.ds(w * CPT, CPT)], idst)

    def body(j, carry):
        pltpu.async_copy(y_hbm.at[isrc.at[j]], rows, sem).wait()
        pltpu.sync_copy(rows, agg_s.at[idst.at[j]], add=True)
        return carry

    lax.fori_loop(0, CPT, body, 0)
    plsc.subcore_barrier()

    for k in range(SLAB // CHUNK):
        pltpu.sync_copy(agg_s.at[pl.ds(s * SLAB + k * CHUNK, CHUNK)], stage)
        pltpu.sync_copy(
            stage, out_hbm.at[pl.ds(c * NPAD + s * SLAB + k * CHUNK, CHUNK)])


def _tc_linear_body(x_ref, w_ref, d0_ref, d1_ref, y_ref):
    xw = jnp.dot(x_ref[...], w_ref[...], preferred_element_type=jnp.float32)
    deg = d0_ref[...] + d1_ref[...] + 1.0
    y_ref[...] = xw * lax.rsqrt(deg)


def _tc_linear(x, w_pad, d0, d1):
    blk = 1000
    return pl.pallas_call(
        _tc_linear_body,
        grid=(N_NODES // blk,),
        in_specs=[
            pl.BlockSpec((blk, IN_C), lambda i: (i, 0)),
            pl.BlockSpec((IN_C, D), lambda i: (0, 0)),
            pl.BlockSpec((blk, 1), lambda i: (i, 0)),
            pl.BlockSpec((blk, 1), lambda i: (i, 0)),
        ],
        out_specs=pl.BlockSpec((blk, D), lambda i: (i, 0)),
        out_shape=jax.ShapeDtypeStruct((N_NODES, D), jnp.float32),
    )(x, w_pad, d0, d1)


def _tc_combine_body(a0_ref, a1_ref, y_ref, d0_ref, d1_ref, b_ref, o_ref):
    dinv = lax.rsqrt(d0_ref[...] + d1_ref[...] + 1.0)
    o_ref[...] = (a0_ref[...] + a1_ref[...] + y_ref[...]) * dinv + b_ref[...]


def _tc_combine(a0, a1, y, d0, d1, b_pad):
    blk = 1000
    return pl.pallas_call(
        _tc_combine_body,
        grid=(N_NODES // blk,),
        in_specs=[
            pl.BlockSpec((blk, D), lambda i: (i, 0)),
            pl.BlockSpec((blk, D), lambda i: (i, 0)),
            pl.BlockSpec((blk, D), lambda i: (i, 0)),
            pl.BlockSpec((blk, 1), lambda i: (i, 0)),
            pl.BlockSpec((blk, 1), lambda i: (i, 0)),
            pl.BlockSpec((1, D), lambda i: (0, 0)),
        ],
        out_specs=pl.BlockSpec((blk, D), lambda i: (i, 0)),
        out_shape=jax.ShapeDtypeStruct((N_NODES, D), jnp.float32),
    )(a0, a1, y, d0, d1, b_pad)


def kernel(x, edge_index, W, b):
    src = edge_index[0].astype(jnp.int32)
    dst = edge_index[1].astype(jnp.int32)

    npad_e = EPAD - N_EDGES
    pad_src = (jnp.arange(npad_e, dtype=jnp.int32) * 131) % N_NODES
    pad_dst = N_NODES + (jnp.arange(npad_e, dtype=jnp.int32) % (NPAD - N_NODES))
    src2d = jnp.concatenate([src, pad_src]).reshape(CT, CHUNK)
    dst2d = jnp.concatenate([dst, pad_dst]).reshape(CT, CHUNK)

    deg_flat = _sc_degree(dst2d)
    d0 = deg_flat[:N_NODES].reshape(N_NODES, 1)
    d1 = deg_flat[NPAD:NPAD + N_NODES].reshape(N_NODES, 1)

    w_pad = jnp.pad(W, ((0, 0), (0, D - W.shape[1])))
    y = _tc_linear(x, w_pad, d0, d1)

    agg_flat = _sc_aggregate(y, src2d, dst2d)
    a0 = agg_flat[:N_NODES]
    a1 = agg_flat[NPAD:NPAD + N_NODES]

    b_pad = jnp.pad(b, (0, D - b.shape[0])).reshape(1, D)
    out = _tc_combine(a0, a1, y, d0, d1, b_pad)
    return out[:, :b.shape[0]]


# speedup vs baseline: 66.7707x; 1.5547x over previous
import functools

import jax
import jax.numpy as jnp
from jax import lax
from jax.experimental import pallas as pl
from jax.experimental.pallas import tpu as pltpu
from jax.experimental.pallas import tpu_sc as plsc

N_NODES = 10000
N_EDGES = 320000
IN_C = 128
OUT_C = 10
D = 16
NPAD = 10240
SLAB = NPAD // 16
CHUNK = 128
CPT = 80
NTILES = 32
CT = CPT * NTILES
EPAD = CT * CHUNK
K_OUT = 8
NBUF = 4

_mesh = plsc.VectorSubcoreMesh(core_axis_name="c", subcore_axis_name="s")
_sc_params = pltpu.CompilerParams(use_tc_tiling_on_sc=False)


@functools.partial(
    pl.kernel,
    mesh=_mesh,
    out_type=jax.ShapeDtypeStruct((2 * NPAD,), jnp.float32),
    compiler_params=_sc_params,
    scratch_types=[
        pltpu.VMEM((CPT, CHUNK), jnp.int32),
        pltpu.VMEM((CHUNK,), jnp.float32),
        pltpu.VMEM((SLAB,), jnp.float32),
        pltpu.VMEM_SHARED((NPAD,), jnp.float32),
        pltpu.SemaphoreType.DMA,
    ],
)
def _sc_degree(dst_hbm, out_hbm, idx_v, ones_v, stage_v, hist_s, sem):
    c = lax.axis_index("c")
    s = lax.axis_index("s")
    w = c * 16 + s

    for i in range(CHUNK // 16):
        ones_v[pl.ds(i * 16, 16)] = jnp.ones((16,), jnp.float32)
    for i in range(SLAB // 16):
        stage_v[pl.ds(i * 16, 16)] = jnp.zeros((16,), jnp.float32)

    pltpu.sync_copy(stage_v, hist_s.at[pl.ds(s * SLAB, SLAB)])
    plsc.subcore_barrier()

    pltpu.sync_copy(dst_hbm.at[pl.ds(w * CPT, CPT)], idx_v)

    def body(j, carry):
        @pl.when(j >= K_OUT)
        def _():
            pltpu.make_async_copy(
                ones_v, hist_s.at[idx_v.at[j]], sem).wait()
        pltpu.async_copy(ones_v, hist_s.at[idx_v.at[j]], sem, add=True)
        return carry

    lax.fori_loop(0, CPT, body, 0)
    for _ in range(K_OUT):
        pltpu.make_async_copy(ones_v, hist_s.at[pl.ds(0, CHUNK)], sem).wait()
    plsc.subcore_barrier()

    pltpu.sync_copy(hist_s.at[pl.ds(s * SLAB, SLAB)], stage_v)
    pltpu.sync_copy(stage_v, out_hbm.at[pl.ds(c * NPAD + s * SLAB, SLAB)])


@functools.partial(
    pl.kernel,
    mesh=_mesh,
    out_type=jax.ShapeDtypeStruct((2 * NPAD, D), jnp.float32),
    compiler_params=_sc_params,
    scratch_types=[
        pltpu.VMEM((CPT, CHUNK), jnp.int32),
        pltpu.VMEM((CPT, CHUNK), jnp.int32),
        pltpu.VMEM((NBUF, CHUNK, D), jnp.float32),
        pltpu.VMEM((CHUNK, D), jnp.float32),
        pltpu.VMEM_SHARED((NPAD, D), jnp.float32),
        pltpu.SemaphoreType.DMA,
        pltpu.SemaphoreType.DMA,
        pltpu.SemaphoreType.DMA,
        pltpu.SemaphoreType.DMA,
    ],
)
def _sc_aggregate(y_hbm, src_hbm, dst_hbm, out_hbm,
                  isrc, idst, ring, stage, agg_s, g0, g1, g2, g3):
    c = lax.axis_index("c")
    s = lax.axis_index("s")
    w = c * 16 + s
    gsems = (g0, g1, g2, g3)

    for i in range(CHUNK):
        stage[i] = jnp.zeros((16,), jnp.float32)
    for k in range(SLAB // CHUNK):
        pltpu.sync_copy(stage, agg_s.at[pl.ds(s * SLAB + k * CHUNK, CHUNK)])
    plsc.subcore_barrier()

    pltpu.sync_copy(src_hbm.at[pl.ds(w * CPT, CPT)], isrc)
    pltpu.sync_copy(dst_hbm.at[pl.ds(w * CPT, CPT)], idst)

    for b in range(NBUF):
        pltpu.async_copy(y_hbm.at[isrc.at[b]], ring.at[b], gsems[b])

    def group(g, carry):
        for b in range(NBUF):
            j = g * NBUF + b
            pltpu.make_async_copy(
                y_hbm.at[isrc.at[j]], ring.at[b], gsems[b]).wait()
            pltpu.sync_copy(ring.at[b], agg_s.at[idst.at[j]], add=True)

            @pl.when(j + NBUF < CPT)
            def _():
                pltpu.async_copy(
                    y_hbm.at[isrc.at[j + NBUF]], ring.at[b], gsems[b])
        return carry

    lax.fori_loop(0, CPT // NBUF, group, 0)
    plsc.subcore_barrier()

    for k in range(SLAB // CHUNK):
        pltpu.sync_copy(agg_s.at[pl.ds(s * SLAB + k * CHUNK, CHUNK)], stage)
        pltpu.sync_copy(
            stage, out_hbm.at[pl.ds(c * NPAD + s * SLAB + k * CHUNK, CHUNK)])


def _tc_linear_body(x_ref, w_ref, d0_ref, d1_ref, y_ref, dinv_ref):
    xw = jnp.dot(x_ref[...], w_ref[...], preferred_element_type=jnp.float32)
    dinv = lax.rsqrt(d0_ref[...] + d1_ref[...] + 1.0)
    dinv_ref[...] = dinv
    y_ref[...] = xw * dinv


def _tc_linear(x, w_pad, d0, d1):
    blk = 1000
    return pl.pallas_call(
        _tc_linear_body,
        grid=(N_NODES // blk,),
        in_specs=[
            pl.BlockSpec((blk, IN_C), lambda i: (i, 0)),
            pl.BlockSpec((IN_C, D), lambda i: (0, 0)),
            pl.BlockSpec((blk, 1), lambda i: (i, 0)),
            pl.BlockSpec((blk, 1), lambda i: (i, 0)),
        ],
        out_specs=[
            pl.BlockSpec((blk, D), lambda i: (i, 0)),
            pl.BlockSpec((blk, 1), lambda i: (i, 0)),
        ],
        out_shape=[
            jax.ShapeDtypeStruct((N_NODES, D), jnp.float32),
            jax.ShapeDtypeStruct((N_NODES, 1), jnp.float32),
        ],
    )(x, w_pad, d0, d1)


def _tc_combine_body(agg_ref, y_ref, dinv_ref, b_ref, o_ref):
    a0 = agg_ref[pl.ds(0, N_NODES), :]
    a1 = agg_ref[pl.ds(NPAD, N_NODES), :]
    res = (a0 + a1 + y_ref[...]) * dinv_ref[...] + b_ref[...]
    o_ref[...] = res[:, :OUT_C]


def _tc_combine(agg_flat, y, dinv, b_pad):
    return pl.pallas_call(
        _tc_combine_body,
        out_shape=jax.ShapeDtypeStruct((N_NODES, OUT_C), jnp.float32),
    )(agg_flat, y, dinv, b_pad)


def kernel(x, edge_index, W, b):
    src = edge_index[0].astype(jnp.int32)
    dst = edge_index[1].astype(jnp.int32)

    npad_e = EPAD - N_EDGES
    pad_src = (jnp.arange(npad_e, dtype=jnp.int32) * 131) % N_NODES
    pad_dst = N_NODES + (jnp.arange(npad_e, dtype=jnp.int32) % (NPAD - N_NODES))
    src2d = jnp.concatenate([src, pad_src]).reshape(CT, CHUNK)
    dst2d = jnp.concatenate([dst, pad_dst]).reshape(CT, CHUNK)

    deg_flat = _sc_degree(dst2d)
    d0 = deg_flat[:N_NODES].reshape(N_NODES, 1)
    d1 = deg_flat[NPAD:NPAD + N_NODES].reshape(N_NODES, 1)

    w_pad = jnp.pad(W, ((0, 0), (0, D - W.shape[1])))
    y, dinv = _tc_linear(x, w_pad, d0, d1)

    agg_flat = _sc_aggregate(y, src2d, dst2d)

    b_pad = jnp.pad(b, (0, D - b.shape[0])).reshape(1, D)
    return _tc_combine(agg_flat, y, dinv, b_pad)
